# baseline (speedup 1.0000x reference)
import jax
import jax.numpy as jnp
from jax.experimental import pallas as pl
from jax.experimental.pallas import tpu as pltpu

B = 256


def _mlp_head_kernel(xg_ref, xpg_ref, wg1_ref, bg1_ref, wl1_ref, bl1_ref,
                     wf1_ref, bf1_ref, wf2_ref, bf2_ref, wo_ref, bo_ref,
                     out_ref):
    xg = jnp.maximum(
        jnp.dot(xg_ref[...], wg1_ref[...], preferred_element_type=jnp.float32)
        + bg1_ref[...][None, :], 0.0)
    xpg = jnp.maximum(
        jnp.dot(xpg_ref[...], wl1_ref[...], preferred_element_type=jnp.float32)
        + bl1_ref[...][None, :], 0.0)
    xc = jnp.concatenate([xg, xpg], axis=1)
    xc = jnp.maximum(
        jnp.dot(xc, wf1_ref[...], preferred_element_type=jnp.float32)
        + bf1_ref[...][None, :], 0.0)
    xc = jnp.maximum(
        jnp.dot(xc, wf2_ref[...], preferred_element_type=jnp.float32)
        + bf2_ref[...][None, :], 0.0)
    out_ref[...] = (
        jnp.dot(xc, wo_ref[...], preferred_element_type=jnp.float32)
        + bo_ref[...][None, :])


def _mlp_head(xg, xpg, Wg1, bg1, Wl1, bl1, Wf1, bf1, Wf2, bf2, Wo, bo):
    return pl.pallas_call(
        _mlp_head_kernel,
        out_shape=jax.ShapeDtypeStruct((B, 1), jnp.float32),
    )(xg, xpg, Wg1, bg1, Wl1, bl1, Wf1, bf1, Wf2, bf2, Wo, bo)


def _gat_conv(x, edge_index, W, a_src, a_dst, bias, negative_slope=0.2):
    N = x.shape[0]
    H, C = a_src.shape
    loop = jnp.arange(N)
    src = jnp.concatenate([edge_index[0], loop])
    dst = jnp.concatenate([edge_index[1], loop])
    xw = (x @ W).reshape(N, H, C)
    alpha_src = jnp.sum(xw * a_src[None, :, :], axis=-1)
    alpha_dst = jnp.sum(xw * a_dst[None, :, :], axis=-1)
    alpha = alpha_src[src] + alpha_dst[dst]
    alpha = jax.nn.leaky_relu(alpha, negative_slope)
    amax = jax.ops.segment_max(alpha, dst, num_segments=N)
    alpha = jnp.exp(alpha - amax[dst])
    denom = jax.ops.segment_sum(alpha, dst, num_segments=N)
    alpha = alpha / (denom[dst] + 1e-16)
    out = jax.ops.segment_sum(xw[src] * alpha[:, :, None], dst, num_segments=N)
    return out.reshape(N, H * C) + bias


def _gcn_conv(x, edge_index, W, bias):
    N = x.shape[0]
    loop = jnp.arange(N)
    src = jnp.concatenate([edge_index[0], loop])
    dst = jnp.concatenate([edge_index[1], loop])
    deg = jax.ops.segment_sum(jnp.ones(src.shape[0], jnp.float32), dst, num_segments=N)
    dinv = jax.lax.rsqrt(jnp.maximum(deg, 1.0))
    norm = dinv[src] * dinv[dst]
    xw = x @ W
    out = jax.ops.segment_sum(xw[src] * norm[:, None], dst, num_segments=N)
    return out + bias


def kernel(x_drug, edge_index_drug, batch_drug, x_prots, edge_index_prots, batch_prots, W1, a_src1, a_dst1, b1, W2, a_src2, a_dst2, b2, Wg1, bg1, Wp1, bp1, Wp2, bp2, Wl1, bl1, Wf1, bf1, Wf2, bf2, Wo, bo):
    x = jax.nn.elu(_gat_conv(x_drug, edge_index_drug, W1, a_src1, a_dst1, b1))
    x = _gat_conv(x, edge_index_drug, W2, a_src2, a_dst2, b2)
    x = jax.nn.relu(x)
    xg = jax.ops.segment_max(x, batch_drug, num_segments=B)
    xp = jax.nn.relu(_gcn_conv(x_prots, edge_index_prots, Wp1, bp1))
    xp = jax.nn.relu(_gcn_conv(xp, edge_index_prots, Wp2, bp2))
    xpg = jax.ops.segment_max(xp, batch_prots, num_segments=B)
    return _mlp_head(xg, xpg, Wg1, bg1, Wl1, bl1, Wf1, bf1, Wf2, bf2, Wo, bo)


# speedup vs baseline: 1.9501x; 1.9501x over previous
import functools

import jax
import jax.numpy as jnp
from jax import lax
from jax.experimental import pallas as pl
from jax.experimental.pallas import tpu as pltpu
from jax.experimental.pallas import tpu_sc as plsc

B = 256
NC, NS, LANES = 2, 16, 16
NW = NC * NS


def _make_seg_agg(n_rows, n_pad, ept, ck, dv, woff, wstride):
    dvp = -(-dv // 8) * 8
    D = dvp * 16
    nchunks = ept // ck
    mesh = plsc.VectorSubcoreMesh(core_axis_name="c", subcore_axis_name="s")

    @functools.partial(
        pl.kernel, mesh=mesh,
        out_type=jax.ShapeDtypeStruct((n_rows, D), jnp.float32),
        scratch_types=[
            pltpu.VMEM((ck,), jnp.int32),
            pltpu.VMEM((ck, D), jnp.float32),
            pltpu.VMEM((ck * wstride,), jnp.float32),
            pltpu.VMEM((D,), jnp.float32),
            pltpu.VMEM((16,), jnp.int32),
            pltpu.SemaphoreType.DMA,
        ],
    )
    def kern(table, w, src, dst, out, srcb, rowsb, wbuf, acc, dstb, sem):
        wid = lax.axis_index("s") * NC + lax.axis_index("c")
        e0 = wid * ept
        stage0 = n_pad + 2 * wid
        stage1 = stage0 + 1

        pltpu.sync_copy(dst.at[pl.ds(e0, 16)], dstb)
        first_dst = dstb[...][0]

        def zero_acc():
            for v in range(dv):
                acc[pl.ds(v * 16, 16)] = jnp.zeros((16,), jnp.float32)

        for v in range(dvp):
            acc[pl.ds(v * 16, 16)] = jnp.zeros((16,), jnp.float32)
        pltpu.sync_copy(acc, out.at[stage0])

        def chunk(c, cur):
            base = e0 + c * ck
            pltpu.sync_copy(src.at[pl.ds(base, ck)], srcb)
            pltpu.sync_copy(dst.at[pl.ds(base, ck)], dstb.at[pl.ds(0, ck)])
            pltpu.sync_copy(w.at[pl.ds(base * wstride, ck * wstride)], wbuf)
            pltpu.async_copy(table.at[srcb], rowsb, sem).wait()
            dvec = dstb[...]
            for j in range(ck):
                dj = dvec[j]
                flush = dj != cur

                @pl.when(flush)
                def _():
                    row = jnp.where(cur == first_dst, stage0, cur)
                    pltpu.sync_copy(acc, out.at[row])
                    zero_acc()

                cur = jnp.where(flush, dj, cur)
                for v in range(dv):
                    wv = wbuf[pl.ds(j * wstride + woff[v], 16)]
                    sl = pl.ds(v * 16, 16)
                    acc[sl] = acc[sl] + rowsb[j, sl] * wv
            return cur

        cur = lax.fori_loop(0, nchunks, chunk, first_dst)
        pltpu.sync_copy(acc, out.at[stage1])

    return kern


def _prep_graph(edge_index, N, ck):
    src = jnp.concatenate(
        [edge_index[0].astype(jnp.int32), jnp.arange(N, dtype=jnp.int32)])
    dst = jnp.concatenate(
        [edge_index[1].astype(jnp.int32), jnp.arange(N, dtype=jnp.int32)])
    order = jnp.argsort(dst)
    src_s = src[order]
    dst_s = dst[order]
    E = int(src_s.shape[0])
    ept = -(-E // (NW * ck)) * ck
    e_pad = NW * ept
    src_p = jnp.concatenate(
        [src_s, jnp.zeros((e_pad - E,), jnp.int32)])
    dst_p = jnp.concatenate(
        [dst_s, jnp.full((e_pad - E,), N, jnp.int32)])
    tiles = dst_p.reshape(NW, ept)
    fd = tiles[:, 0]
    ld = tiles[:, -1]
    ids = jnp.stack([fd, ld], axis=1).reshape(-1)
    deg = (jnp.searchsorted(dst_s, jnp.arange(1, N + 1, dtype=jnp.int32)) -
           jnp.searchsorted(dst_s, jnp.arange(N, dtype=jnp.int32)))
    return dict(src_s=src_s, dst_s=dst_s, src_p=src_p, dst_p=dst_p, fd=fd,
                ids=ids, deg=deg.astype(jnp.float32), ept=ept, E=E, N=N)


def _seg_aggregate(table, wrow, g, dv, woff, ck):
    N = g["N"]
    n_pad = N + 8
    n_rows = n_pad + 2 * NW
    e_pad = g["src_p"].shape[0]
    wstride = wrow.shape[1]
    wp = jnp.zeros((e_pad, wstride), jnp.float32).at[:g["E"]].set(
        wrow).reshape(-1)
    dvp = -(-dv // 8) * 8
    if dvp * 16 > table.shape[1]:
        table = jnp.concatenate(
            [table, jnp.zeros((N, dvp * 16 - table.shape[1]), jnp.float32)],
            axis=1)
    kern = _make_seg_agg(n_rows, n_pad, g["ept"], ck, dv, woff, wstride)
    out_full = kern(table, wp, g["src_p"], g["dst_p"])
    stage = out_full[n_pad:n_pad + 2 * NW]
    body = out_full[:N + 1]
    body = body.at[g["ids"]].set(0.0)
    body = body.at[g["ids"]].add(stage)
    return body[:N]


def _mm_kernel(x_ref, w_ref, o_ref):
    o_ref[...] = jnp.dot(x_ref[...], w_ref[...],
                         preferred_element_type=jnp.float32)


def _matmul(x, W, blk=2000):
    N, K = x.shape
    M = W.shape[1]
    return pl.pallas_call(
        _mm_kernel,
        grid=(N // blk,),
        in_specs=[
            pl.BlockSpec((blk, K), lambda i: (i, 0)),
            pl.BlockSpec((K, M), lambda i: (0, 0)),
        ],
        out_specs=pl.BlockSpec((blk, M), lambda i: (i, 0)),
        out_shape=jax.ShapeDtypeStruct((N, M), jnp.float32),
    )(x, W)


def _mlp_head_kernel(xg_ref, xpg_ref, wg1_ref, bg1_ref, wl1_ref, bl1_ref,
                     wf1_ref, bf1_ref, wf2_ref, bf2_ref, wo_ref, bo_ref,
                     out_ref):
    xg = jnp.maximum(
        jnp.dot(xg_ref[...], wg1_ref[...], preferred_element_type=jnp.float32)
        + bg1_ref[...][None, :], 0.0)
    xpg = jnp.maximum(
        jnp.dot(xpg_ref[...], wl1_ref[...], preferred_element_type=jnp.float32)
        + bl1_ref[...][None, :], 0.0)
    xc = jnp.concatenate([xg, xpg], axis=1)
    xc = jnp.maximum(
        jnp.dot(xc, wf1_ref[...], preferred_element_type=jnp.float32)
        + bf1_ref[...][None, :], 0.0)
    xc = jnp.maximum(
        jnp.dot(xc, wf2_ref[...], preferred_element_type=jnp.float32)
        + bf2_ref[...][None, :], 0.0)
    out_ref[...] = (
        jnp.dot(xc, wo_ref[...], preferred_element_type=jnp.float32)
        + bo_ref[...][None, :])


def _mlp_head(xg, xpg, Wg1, bg1, Wl1, bl1, Wf1, bf1, Wf2, bf2, Wo, bo):
    return pl.pallas_call(
        _mlp_head_kernel,
        out_shape=jax.ShapeDtypeStruct((B, 1), jnp.float32),
    )(xg, xpg, Wg1, bg1, Wl1, bl1, Wf1, bf1, Wf2, bf2, Wo, bo)


def _gat_conv(x, g, W, a_src, a_dst, bias, ck, negative_slope=0.2):
    N = x.shape[0]
    H, C = a_src.shape
    Cp = -(-C // 16) * 16
    xw = _matmul(x, W)
    xwh = xw.reshape(N, H, C)
    s = jnp.sum(xwh * a_src[None], axis=-1)
    d = jnp.sum(xwh * a_dst[None], axis=-1)
    alpha = s[g["src_s"]] + d[g["dst_s"]]
    alpha = jax.nn.leaky_relu(alpha, negative_slope)
    we = jnp.exp(alpha)
    wb = jnp.repeat(we, 16, axis=1)
    w16 = jnp.zeros((g["E"], 16), jnp.float32).at[:, :H].set(we)
    wrow = jnp.concatenate([wb, w16], axis=1)
    tab = jnp.zeros((N, H, Cp), jnp.float32).at[:, :, :C].set(xwh)
    tab = jnp.concatenate(
        [tab.reshape(N, H * Cp), jnp.ones((N, 16), jnp.float32)], axis=1)
    dv = (H * Cp) // 16 + 1
    woff = ([min(v // (Cp // 16), H - 1) * 16 for v in range(dv - 1)]
            + [H * 16])
    agg = _seg_aggregate(tab, wrow, g, dv, woff, ck)
    denom = agg[:, H * Cp:H * Cp + H]
    feats = agg[:, :H * Cp].reshape(N, H, Cp)[:, :, :C]
    out = feats / (denom[:, :, None] + 1e-16)
    return out.reshape(N, H * C) + bias


def _gcn_conv(x, g, W, bias, ck):
    N = x.shape[0]
    M = W.shape[1]
    Mp = -(-M // 16) * 16
    xw = _matmul(x, W)
    dinv = lax.rsqrt(jnp.maximum(g["deg"], 1.0))
    we = dinv[g["src_s"]] * dinv[g["dst_s"]]
    wrow = jnp.repeat(we[:, None], 16, axis=1)
    tab = jnp.zeros((N, Mp), jnp.float32).at[:, :M].set(xw)
    dv = Mp // 16
    agg = _seg_aggregate(tab, wrow, g, dv, [0] * dv, ck)
    return agg[:, :M] + bias


def kernel(x_drug, edge_index_drug, batch_drug, x_prots, edge_index_prots,
           batch_prots, W1, a_src1, a_dst1, b1, W2, a_src2, a_dst2, b2,
           Wg1, bg1, Wp1, bp1, Wp2, bp2, Wl1, bl1, Wf1, bf1, Wf2, bf2,
           Wo, bo):
    gd = _prep_graph(edge_index_drug, x_drug.shape[0], 8)
    gp = _prep_graph(edge_index_prots, x_prots.shape[0], 16)

    x = jax.nn.elu(_gat_conv(x_drug, gd, W1, a_src1, a_dst1, b1, 8))
    x = jax.nn.relu(_gat_conv(x, gd, W2, a_src2, a_dst2, b2, 8))
    xg = jax.ops.segment_max(x, batch_drug, num_segments=B)

    xp = jax.nn.relu(_gcn_conv(x_prots, gp, Wp1, bp1, 16))
    xp = jax.nn.relu(_gcn_conv(xp, gp, Wp2, bp2, 16))
    xpg = jax.ops.segment_max(xp, batch_prots, num_segments=B)

    return _mlp_head(xg, xpg, Wg1, bg1, Wl1, bl1, Wf1, bf1, Wf2, bf2, Wo, bo)
